# baseline (speedup 1.0000x reference)
import functools

import jax
import jax.numpy as jnp
from jax import lax
from jax.experimental import pallas as pl
from jax.experimental.pallas import tpu as pltpu
from jax.experimental.pallas import tpu_sc as plsc

_NUM_EMB = 1024
_EMB = 8
_ROWS = 16384
_COLS = 200
_TOTAL = _ROWS * _COLS
_NC = 2
_NS = 16
_NW = _NC * _NS
_JT = _COLS // 8
_IT = _ROWS // 128
_IT_W = _IT // _NW
_CHUNK = _IT_W * 8 * 128
_OUT_WORDS = _CHUNK * _EMB
_GROUPS = _CHUNK // 16

_mesh = plsc.VectorSubcoreMesh(core_axis_name="c", subcore_axis_name="s")


@functools.partial(
    pl.kernel,
    out_type=jax.ShapeDtypeStruct((_TOTAL * _EMB,), jnp.float32),
    mesh=_mesh,
    scratch_types=[
        pltpu.VMEM((_NUM_EMB * _EMB,), jnp.float32),
        pltpu.VMEM((_CHUNK,), jnp.int32),
        pltpu.VMEM((_CHUNK,), jnp.int32),
        pltpu.VMEM((_OUT_WORDS,), jnp.float32),
        pltpu.VMEM((_OUT_WORDS,), jnp.float32),
        pltpu.SemaphoreType.DMA,
        pltpu.SemaphoreType.DMA,
        pltpu.SemaphoreType.DMA,
        pltpu.SemaphoreType.DMA,
    ],
    compiler_params=pltpu.CompilerParams(needs_layout_passes=False),
)
def _emb_lookup(keys_hbm, table_hbm, out_hbm,
                table_v, keys0, keys1, out0, out1,
                sin0, sin1, sout0, sout1):
    wid = lax.axis_index("s") * _NC + lax.axis_index("c")
    it0 = wid * _IT_W

    pltpu.sync_copy(table_hbm, table_v)

    kbufs = (keys0, keys1)
    obufs = (out0, out1)
    sins = (sin0, sin1)
    souts = (sout0, sout1)

    def start_in(jt):
        b = jt & 1
        off = (jt * _IT) * 1024 + it0 * 1024
        return pltpu.async_copy(
            keys_hbm.at[pl.ds(off, _CHUNK)], kbufs[b], sins[b])

    in_h = {0: start_in(0)}
    out_h = {}

    def compute(jt):
        kv = kbufs[jt & 1]
        ov = obufs[jt & 1]

        @plsc.parallel_loop(0, 8 * _IT_W * 8, unroll=8)
        def body(p):
            jr = p >> 5
            q = p & 31
            t = q >> 3
            g = q & 7
            koff = t * 1024 + jr * 128 + g * 16
            obase = jr * (_IT_W * 1024) + t * 1024 + g * 16
            k16 = kv[pl.ds(koff, 16)]
            k8 = k16 * _EMB
            for k in range(_EMB):
                col = plsc.load_gather(table_v, [k8 + k])
                ov[pl.ds(obase + k * 128, 16)] = col

    for jt in range(_JT):
        b = jt & 1
        if jt + 1 < _JT:
            in_h[jt + 1] = start_in(jt + 1)
        in_h.pop(jt).wait()
        if jt - 2 in out_h:
            for h in out_h.pop(jt - 2):
                h.wait()
        compute(jt)
        hs = []
        for jr in range(8):
            off = (jt * 8 + jr) * (_IT * 1024) + it0 * 1024
            hs.append(pltpu.async_copy(
                obufs[b].at[pl.ds(jr * (_IT_W * 1024), _IT_W * 1024)],
                out_hbm.at[pl.ds(off, _IT_W * 1024)],
                souts[b]))
        out_h[jt] = hs

    for jt in sorted(out_h):
        for h in out_h.pop(jt):
            h.wait()


def kernel(keys, table):
    keys_flat = (keys.astype(jnp.int32)
                 .T.reshape(_JT, 8, _IT, 128)
                 .transpose(0, 2, 1, 3)
                 .reshape(-1))
    table_flat = table.reshape(-1)
    out = _emb_lookup(keys_flat, table_flat)
    return (out.reshape(_COLS, _IT, _EMB, 128)
            .transpose(1, 3, 0, 2)
            .reshape(_ROWS, _COLS, _EMB))


# speedup vs baseline: 1.9271x; 1.9271x over previous
import functools

import jax
import jax.numpy as jnp
from jax import lax
from jax.experimental import pallas as pl
from jax.experimental.pallas import tpu as pltpu
from jax.experimental.pallas import tpu_sc as plsc

_NUM_EMB = 1024
_EMB = 8
_ROWS = 16384
_COLS = 200
_TOTAL = _ROWS * _COLS
_NC = 2
_NS = 16
_NW = _NC * _NS
_JT = _COLS // 8
_IT = _ROWS // 128
_IT_W = _IT // _NW
_CHUNK = _IT_W * 8 * 128
_OUT_WORDS = _CHUNK * _EMB
_GROUPS = _CHUNK // 16

_mesh = plsc.VectorSubcoreMesh(core_axis_name="c", subcore_axis_name="s")


@functools.partial(
    pl.kernel,
    out_type=jax.ShapeDtypeStruct((_TOTAL * _EMB,), jnp.float32),
    mesh=_mesh,
    scratch_types=[
        pltpu.VMEM((_NUM_EMB * 9,), jnp.float32),
        pltpu.VMEM((_CHUNK,), jnp.int32),
        pltpu.VMEM((_CHUNK,), jnp.int32),
        pltpu.VMEM((_OUT_WORDS,), jnp.float32),
        pltpu.VMEM((_OUT_WORDS,), jnp.float32),
        pltpu.SemaphoreType.DMA,
        pltpu.SemaphoreType.DMA,
        pltpu.SemaphoreType.DMA,
        pltpu.SemaphoreType.DMA,
    ],
    compiler_params=pltpu.CompilerParams(needs_layout_passes=False),
)
def _emb_lookup(keys_hbm, table_hbm, out_hbm,
                table_v, keys0, keys1, out0, out1,
                sin0, sin1, sout0, sout1):
    wid = lax.axis_index("s") * _NC + lax.axis_index("c")
    it0 = wid * _IT_W

    pltpu.sync_copy(table_hbm, table_v)

    kbufs = (keys0, keys1)
    obufs = (out0, out1)
    sins = (sin0, sin1)
    souts = (sout0, sout1)

    def start_in(jt):
        b = jt & 1
        off = (jt * _IT) * 1024 + it0 * 1024
        return pltpu.async_copy(
            keys_hbm.at[pl.ds(off, _CHUNK)], kbufs[b], sins[b])

    in_h = {0: start_in(0)}
    out_h = {}

    def compute(jt):
        kv = kbufs[jt & 1]
        ov = obufs[jt & 1]

        @plsc.parallel_loop(0, 8 * _IT_W * 8, unroll=4)
        def body(p):
            jr = p >> 5
            q = p & 31
            t = q >> 3
            g = q & 7
            koff = t * 1024 + jr * 128 + g * 16
            obase = jr * (_IT_W * 1024) + t * 1024 + g * 16
            k16 = kv[pl.ds(koff, 16)]
            k9 = k16 * 9
            for k in range(_EMB):
                col = plsc.load_gather(table_v, [k9 + k])
                ov[pl.ds(obase + k * 128, 16)] = col

    for jt in range(_JT):
        b = jt & 1
        if jt + 1 < _JT:
            in_h[jt + 1] = start_in(jt + 1)
        in_h.pop(jt).wait()
        if jt - 2 in out_h:
            for h in out_h.pop(jt - 2):
                h.wait()
        compute(jt)
        hs = []
        for jr in range(8):
            off = (jt * 8 + jr) * (_IT * 1024) + it0 * 1024
            hs.append(pltpu.async_copy(
                obufs[b].at[pl.ds(jr * (_IT_W * 1024), _IT_W * 1024)],
                out_hbm.at[pl.ds(off, _IT_W * 1024)],
                souts[b]))
        out_h[jt] = hs

    for jt in sorted(out_h):
        for h in out_h.pop(jt):
            h.wait()


def kernel(keys, table):
    keys_flat = (keys.astype(jnp.int32)
                 .T.reshape(_JT, 8, _IT, 128)
                 .transpose(0, 2, 1, 3)
                 .reshape(-1))
    table_flat = jnp.concatenate(
        [table, jnp.zeros((_NUM_EMB, 1), jnp.float32)], axis=1).reshape(-1)
    out = _emb_lookup(keys_flat, table_flat)
    return (out.reshape(_COLS, _IT, _EMB, 128)
            .transpose(1, 3, 0, 2)
            .reshape(_ROWS, _COLS, _EMB))


# speedup vs baseline: 1.9478x; 1.0108x over previous
import functools

import jax
import jax.numpy as jnp
from jax import lax
from jax.experimental import pallas as pl
from jax.experimental.pallas import tpu as pltpu
from jax.experimental.pallas import tpu_sc as plsc

_NUM_EMB = 1024
_EMB = 8
_ROWS = 16384
_COLS = 200
_TOTAL = _ROWS * _COLS
_NC = 2
_NS = 16
_NW = _NC * _NS
_JT = _COLS // 8
_IT = _ROWS // 128
_IT_W = _IT // _NW
_CHUNK = _IT_W * 8 * 128
_OUT_WORDS = _CHUNK * _EMB
_GROUPS = _CHUNK // 16

_mesh = plsc.VectorSubcoreMesh(core_axis_name="c", subcore_axis_name="s")


@functools.partial(
    pl.kernel,
    out_type=jax.ShapeDtypeStruct((_TOTAL * _EMB,), jnp.float32),
    mesh=_mesh,
    scratch_types=[
        pltpu.VMEM((_NUM_EMB * 9,), jnp.float32),
        pltpu.VMEM((_NUM_EMB * _EMB,), jnp.float32),
        pltpu.VMEM((_CHUNK,), jnp.int32),
        pltpu.VMEM((_CHUNK,), jnp.int32),
        pltpu.VMEM((_OUT_WORDS,), jnp.float32),
        pltpu.VMEM((_OUT_WORDS,), jnp.float32),
        pltpu.SemaphoreType.DMA,
        pltpu.SemaphoreType.DMA,
        pltpu.SemaphoreType.DMA,
        pltpu.SemaphoreType.DMA,
    ],
    compiler_params=pltpu.CompilerParams(needs_layout_passes=False),
)
def _emb_lookup(keys_hbm, table_hbm, out_hbm,
                table_v, tstage, keys0, keys1, out0, out1,
                sin0, sin1, sout0, sout1):
    wid = lax.axis_index("s") * _NC + lax.axis_index("c")
    it0 = wid * _IT_W


    kbufs = (keys0, keys1)
    obufs = (out0, out1)
    sins = (sin0, sin1)
    souts = (sout0, sout1)

    def start_in(jt):
        b = jt & 1
        off = (jt * _IT) * 1024 + it0 * 1024
        return pltpu.async_copy(
            keys_hbm.at[pl.ds(off, _CHUNK)], kbufs[b], sins[b])

    in_h = {0: start_in(0)}
    out_h = {}

    pltpu.sync_copy(table_hbm, tstage)
    lane8 = lax.iota(jnp.int32, 16) * _EMB
    lane9 = lax.iota(jnp.int32, 16) * 9

    @plsc.parallel_loop(0, _NUM_EMB // 16, unroll=2)
    def spread(r):
        i8 = lane8 + r * (16 * _EMB)
        i9 = lane9 + r * (16 * 9)
        for k in range(_EMB):
            v = plsc.load_gather(tstage, [i8 + k])
            plsc.store_scatter(table_v, [i9 + k], v)

    def compute(jt):
        kv = kbufs[jt & 1]
        ov = obufs[jt & 1]

        @plsc.parallel_loop(0, 8 * _IT_W * 8, unroll=4)
        def body(p):
            jr = p >> 5
            q = p & 31
            t = q >> 3
            g = q & 7
            koff = t * 1024 + jr * 128 + g * 16
            obase = jr * (_IT_W * 1024) + t * 1024 + g * 16
            k16 = kv[pl.ds(koff, 16)]
            k9 = k16 * 9
            for k in range(_EMB):
                col = plsc.load_gather(table_v, [k9 + k])
                ov[pl.ds(obase + k * 128, 16)] = col

    for jt in range(_JT):
        b = jt & 1
        if jt + 1 < _JT:
            in_h[jt + 1] = start_in(jt + 1)
        in_h.pop(jt).wait()
        if jt - 2 in out_h:
            for h in out_h.pop(jt - 2):
                h.wait()
        compute(jt)
        hs = []
        for jr in range(8):
            off = (jt * 8 + jr) * (_IT * 1024) + it0 * 1024
            hs.append(pltpu.async_copy(
                obufs[b].at[pl.ds(jr * (_IT_W * 1024), _IT_W * 1024)],
                out_hbm.at[pl.ds(off, _IT_W * 1024)],
                souts[b]))
        out_h[jt] = hs

    for jt in sorted(out_h):
        for h in out_h.pop(jt):
            h.wait()


def kernel(keys, table):
    keys_flat = (keys.astype(jnp.int32)
                 .T.reshape(_JT, 8, _IT, 128)
                 .transpose(0, 2, 1, 3)
                 .reshape(-1))
    out = _emb_lookup(keys_flat, table.reshape(-1))
    return (out.reshape(_COLS, _IT, _EMB, 128)
            .transpose(1, 3, 0, 2)
            .reshape(_ROWS, _COLS, _EMB))


# speedup vs baseline: 2.0134x; 1.0337x over previous
import functools

import jax
import jax.numpy as jnp
from jax import lax
from jax.experimental import pallas as pl
from jax.experimental.pallas import tpu as pltpu
from jax.experimental.pallas import tpu_sc as plsc

_NUM_EMB = 1024
_EMB = 8
_ROWS = 16384
_COLS = 200
_TOTAL = _ROWS * _COLS
_NC = 2
_NS = 16
_NW = _NC * _NS
_JT = _COLS // 8
_IT = _ROWS // 128
_IT_W = _IT // 4
_CHUNK = _IT_W * 128
_OUT_WORDS = _CHUNK * _EMB

_mesh = plsc.VectorSubcoreMesh(core_axis_name="c", subcore_axis_name="s")


@functools.partial(
    pl.kernel,
    out_type=jax.ShapeDtypeStruct((_TOTAL * _EMB,), jnp.float32),
    mesh=_mesh,
    scratch_types=[
        pltpu.VMEM((_NUM_EMB * 9,), jnp.float32),
        pltpu.VMEM((_NUM_EMB * _EMB,), jnp.float32),
        pltpu.VMEM((_IT_W, 1, 128), jnp.int32),
        pltpu.VMEM((_IT_W, 1, 128), jnp.int32),
        pltpu.VMEM((_OUT_WORDS,), jnp.float32),
        pltpu.VMEM((_OUT_WORDS,), jnp.float32),
        pltpu.SemaphoreType.DMA,
        pltpu.SemaphoreType.DMA,
        pltpu.SemaphoreType.DMA,
        pltpu.SemaphoreType.DMA,
    ],
    compiler_params=pltpu.CompilerParams(needs_layout_passes=False),
)
def _emb_lookup(keys_hbm, table_hbm, out_hbm,
                table_v, tstage, keys0, keys1, out0, out1,
                sin0, sin1, sout0, sout1):
    wid = lax.axis_index("s") * _NC + lax.axis_index("c")
    jr = wid >> 2
    h = wid & 3
    it0 = h * _IT_W

    kbufs = (keys0, keys1)
    obufs = (out0, out1)
    sins = (sin0, sin1)
    souts = (sout0, sout1)

    def start_in(jt):
        b = jt & 1
        return pltpu.async_copy(
            keys_hbm.at[pl.ds(jt * _IT + it0, _IT_W), pl.ds(jr, 1), :],
            kbufs[b], sins[b])

    in_h = {0: start_in(0)}
    out_h = {}

    pltpu.sync_copy(table_hbm, tstage)
    lane8 = lax.iota(jnp.int32, 16) * _EMB
    lane9 = lax.iota(jnp.int32, 16) * 9

    @plsc.parallel_loop(0, _NUM_EMB // 16, unroll=2)
    def spread(r):
        i8 = lane8 + r * (16 * _EMB)
        i9 = lane9 + r * (16 * 9)
        for k in range(_EMB):
            v = plsc.load_gather(tstage, [i8 + k])
            plsc.store_scatter(table_v, [i9 + k], v)

    def compute(jt):
        kv = kbufs[jt & 1]
        ov = obufs[jt & 1]

        @plsc.parallel_loop(0, _IT_W * 8, unroll=4)
        def body(p):
            itl = p >> 3
            g = p & 7
            obase = itl * 1024 + g * 16
            k16 = kv[itl, 0, pl.ds(g * 16, 16)]
            k9 = k16 * 9
            for k in range(_EMB):
                col = plsc.load_gather(table_v, [k9 + k])
                ov[pl.ds(obase + k * 128, 16)] = col

    for jt in range(_JT):
        b = jt & 1
        if jt + 1 < _JT:
            in_h[jt + 1] = start_in(jt + 1)
        in_h.pop(jt).wait()
        if jt - 2 in out_h:
            out_h.pop(jt - 2).wait()
        compute(jt)
        off = (jt * 8 + jr) * (_IT * 1024) + it0 * 1024
        out_h[jt] = pltpu.async_copy(
            obufs[b], out_hbm.at[pl.ds(off, _OUT_WORDS)], souts[b])

    for jt in sorted(out_h):
        out_h.pop(jt).wait()


def kernel(keys, table):
    keys3 = (keys.astype(jnp.int32)
             .T.reshape(_JT, 8, _IT, 128)
             .transpose(0, 2, 1, 3)
             .reshape(_JT * _IT, 8, 128))
    out = _emb_lookup(keys3, table.reshape(-1))
    return (out.reshape(_COLS, _IT, _EMB, 128)
            .transpose(1, 3, 0, 2)
            .reshape(_ROWS, _COLS, _EMB))
